# baseline (device time: 30163 ns/iter reference)
import jax
import jax.numpy as jnp
from jax import lax
from jax.experimental import pallas as pl
from jax.experimental.pallas import tpu as pltpu

N_DEV = 4


def kernel(x, router, W1, W2):
    t_loc, d_model = x.shape
    e_loc, _, f_dim = W1.shape
    e_cols = router.shape[1]
    n_exp = N_DEV * e_cols
    t_all = N_DEV * t_loc

    def body(x_ref, r_ref, w1_ref, w2_ref, out_ref,
             xg, rg, wg, og, po, w1v, w2v, wcopy_sem,
             x_send, x_recv, r_send, r_recv,
             w_send, w_recv, o_send, o_recv):
        my = lax.axis_index("i")

        w1_copy = pltpu.make_async_copy(w1_ref, w1v, wcopy_sem.at[0])
        w2_copy = pltpu.make_async_copy(w2_ref, w2v, wcopy_sem.at[1])
        w1_copy.start()
        w2_copy.start()

        barrier = pltpu.get_barrier_semaphore()
        for d in range(1, N_DEV):
            peer = lax.rem(my + d, N_DEV)
            pl.semaphore_signal(barrier, inc=1, device_id=(peer,),
                                device_id_type=pl.DeviceIdType.MESH)
        pl.semaphore_wait(barrier, N_DEV - 1)

        xg[0] = x_ref[...].astype(jnp.bfloat16)
        rg[0] = r_ref[...]

        sends = []

        def bcast(buf, ssem, rsem, d):
            dst = lax.rem(my + (N_DEV - d), N_DEV)
            c = pltpu.make_async_remote_copy(
                src_ref=buf.at[0], dst_ref=buf.at[d],
                send_sem=ssem.at[d], recv_sem=rsem.at[d],
                device_id=(dst,), device_id_type=pl.DeviceIdType.MESH)
            c.start()
            sends.append(c)

        def wait_recv(buf, ssem, rsem, d):
            pltpu.make_async_remote_copy(
                src_ref=buf.at[d], dst_ref=buf.at[d],
                send_sem=ssem.at[d], recv_sem=rsem.at[d],
                device_id=(my,), device_id_type=pl.DeviceIdType.MESH,
            ).wait_recv()

        for d in range(1, N_DEV):
            bcast(rg, r_send, r_recv, d)
        for d in range(1, N_DEV):
            bcast(xg, x_send, x_recv, d)

        for d in range(1, N_DEV):
            wait_recv(rg, r_send, r_recv, d)
        router_rot = jnp.concatenate([rg[d] for d in range(N_DEV)], axis=1)
        gates = jnp.dot(x_ref[...], router_rot,
                        preferred_element_type=jnp.float32,
                        precision=lax.Precision.HIGHEST)
        cols = lax.broadcasted_iota(jnp.int32, gates.shape, 1)
        idx1 = jnp.argmax(gates, axis=1).reshape(t_loc, 1)
        m1 = jnp.max(gates, axis=1, keepdims=True)
        g2 = jnp.where(cols == idx1, -jnp.inf, gates)
        idx2 = jnp.argmax(g2, axis=1).reshape(t_loc, 1)
        m2 = jnp.max(g2, axis=1, keepdims=True)
        b = jnp.exp(m2 - m1)
        wrot = (jnp.where(cols == idx1, 1.0, 0.0)
                + jnp.where(cols == idx2, b, 0.0)) / (1.0 + b)
        wg[0] = wrot.astype(jnp.float32)
        for d in range(1, N_DEV):
            bcast(wg, w_send, w_recv, d)

        for d in range(1, N_DEV):
            wait_recv(xg, x_send, x_recv, d)
        for d in range(1, N_DEV):
            wait_recv(wg, w_send, w_recv, d)

        x_all = xg[...].reshape(t_all, d_model)
        coef = jnp.concatenate(
            [wg[d][:, e_cols * ((N_DEV - d) % N_DEV):
                   e_cols * ((N_DEV - d) % N_DEV) + e_cols]
             for d in range(N_DEV)], axis=0)

        w1_copy.wait()
        w2_copy.wait()

        x32 = x_all.astype(jnp.float32)
        partial = jnp.zeros((t_all, d_model), jnp.float32)
        for e in range(e_loc):
            h = jnp.dot(x32, w1v[e], preferred_element_type=jnp.float32,
                        precision=lax.Precision.DEFAULT)
            h = jnp.maximum(h, 0.0) * coef[:, e:e + 1]
            partial = partial + jnp.dot(h, w2v[e],
                                        preferred_element_type=jnp.float32,
                                        precision=lax.Precision.DEFAULT)

        po[...] = partial.reshape(N_DEV, t_loc, d_model).astype(jnp.bfloat16)
        for d in range(1, N_DEV):
            dst = lax.rem(my + d, N_DEV)
            c = pltpu.make_async_remote_copy(
                src_ref=po.at[d], dst_ref=og.at[N_DEV - d],
                send_sem=o_send.at[d], recv_sem=o_recv.at[N_DEV - d],
                device_id=(dst,), device_id_type=pl.DeviceIdType.MESH)
            c.start()
            sends.append(c)

        for d in range(1, N_DEV):
            wait_recv(og, o_send, o_recv, d)

        acc = partial[0:t_loc, :]
        for d in range(1, N_DEV):
            acc = acc + og[d].astype(jnp.float32)
        out_ref[...] = acc

        for c in sends:
            c.wait_send()

    return pl.pallas_call(
        body,
        out_shape=jax.ShapeDtypeStruct((t_loc, d_model), jnp.float32),
        in_specs=[
            pl.BlockSpec(memory_space=pltpu.VMEM),
            pl.BlockSpec(memory_space=pltpu.VMEM),
            pl.BlockSpec(memory_space=pl.ANY),
            pl.BlockSpec(memory_space=pl.ANY),
        ],
        out_specs=pl.BlockSpec(memory_space=pltpu.VMEM),
        scratch_shapes=[
            pltpu.VMEM((N_DEV, t_loc, d_model), jnp.bfloat16),
            pltpu.VMEM((N_DEV, d_model, e_cols), jnp.float32),
            pltpu.VMEM((N_DEV, t_loc, n_exp), jnp.float32),
            pltpu.VMEM((N_DEV, t_loc, d_model), jnp.bfloat16),
            pltpu.VMEM((N_DEV, t_loc, d_model), jnp.bfloat16),
            pltpu.VMEM((e_loc, d_model, f_dim), jnp.float32),
            pltpu.VMEM((e_loc, f_dim, d_model), jnp.float32),
            pltpu.SemaphoreType.DMA((2,)),
        ] + [pltpu.SemaphoreType.DMA((N_DEV,)) for _ in range(8)],
        compiler_params=pltpu.CompilerParams(collective_id=0),
    )(x, router, W1, W2)


# device time: 27940 ns/iter; 1.0796x vs baseline; 1.0796x over previous
import jax
import jax.numpy as jnp
from jax import lax
from jax.experimental import pallas as pl
from jax.experimental.pallas import tpu as pltpu

N_DEV = 4


def kernel(x, router, W1, W2):
    t_loc, d_model = x.shape
    e_loc, _, f_dim = W1.shape
    e_cols = router.shape[1]
    n_exp = N_DEV * e_cols
    t_all = N_DEV * t_loc

    def body(x_ref, r_ref, w1_ref, w2_ref, out_ref,
             xg, rg, wg, og, po,
             x_send, x_recv, r_send, r_recv,
             w_send, w_recv, o_send, o_recv):
        my = lax.axis_index("i")

        with jax.named_scope("phase_barrier"):
            barrier = pltpu.get_barrier_semaphore()
            for d in range(1, N_DEV):
                peer = lax.rem(my + d, N_DEV)
                pl.semaphore_signal(barrier, inc=1, device_id=(peer,),
                                    device_id_type=pl.DeviceIdType.MESH)
            pl.semaphore_wait(barrier, N_DEV - 1)

        xg[0] = x_ref[...].astype(jnp.bfloat16)
        rg[0] = r_ref[...]

        sends = []

        def bcast(buf, ssem, rsem, d):
            dst = lax.rem(my + (N_DEV - d), N_DEV)
            c = pltpu.make_async_remote_copy(
                src_ref=buf.at[0], dst_ref=buf.at[d],
                send_sem=ssem.at[d], recv_sem=rsem.at[d],
                device_id=(dst,), device_id_type=pl.DeviceIdType.MESH)
            c.start()
            sends.append(c)

        def wait_recv(buf, ssem, rsem, d):
            pltpu.make_async_remote_copy(
                src_ref=buf.at[d], dst_ref=buf.at[d],
                send_sem=ssem.at[d], recv_sem=rsem.at[d],
                device_id=(my,), device_id_type=pl.DeviceIdType.MESH,
            ).wait_recv()

        with jax.named_scope("phase_bcast_start"):
            for d in range(1, N_DEV):
                bcast(rg, r_send, r_recv, d)
            for d in range(1, N_DEV):
                bcast(xg, x_send, x_recv, d)

        with jax.named_scope("phase_router_wait"):
            for d in range(1, N_DEV):
                wait_recv(rg, r_send, r_recv, d)
        ns_gating = jax.named_scope("phase_gating")
        ns_gating.__enter__()
        router_rot = jnp.concatenate([rg[d] for d in range(N_DEV)], axis=1)
        gates = jnp.dot(x_ref[...], router_rot,
                        preferred_element_type=jnp.float32,
                        precision=lax.Precision.HIGHEST)
        cols = lax.broadcasted_iota(jnp.int32, gates.shape, 1)
        idx1 = jnp.argmax(gates, axis=1).reshape(t_loc, 1)
        m1 = jnp.max(gates, axis=1, keepdims=True)
        g2 = jnp.where(cols == idx1, -jnp.inf, gates)
        idx2 = jnp.argmax(g2, axis=1).reshape(t_loc, 1)
        m2 = jnp.max(g2, axis=1, keepdims=True)
        b = jnp.exp(m2 - m1)
        wrot = (jnp.where(cols == idx1, 1.0, 0.0)
                + jnp.where(cols == idx2, b, 0.0)) / (1.0 + b)
        wg[0] = wrot.astype(jnp.float32)
        for d in range(1, N_DEV):
            bcast(wg, w_send, w_recv, d)
        ns_gating.__exit__(None, None, None)

        with jax.named_scope("phase_xw_wait"):
            for d in range(1, N_DEV):
                wait_recv(xg, x_send, x_recv, d)
            for d in range(1, N_DEV):
                wait_recv(wg, w_send, w_recv, d)

        x_all = xg[...].reshape(t_all, d_model)
        coef = jnp.concatenate(
            [wg[d][:, e_cols * ((N_DEV - d) % N_DEV):
                   e_cols * ((N_DEV - d) % N_DEV) + e_cols]
             for d in range(N_DEV)], axis=0)

        with jax.named_scope("phase_ffn"):
            partial = jnp.zeros((t_all, d_model), jnp.float32)
            for e in range(e_loc):
                h = jnp.dot(x_all, w1_ref[e], preferred_element_type=jnp.float32)
                h = jnp.maximum(h, 0.0) * coef[:, e:e + 1]
                partial = partial + jnp.dot(h.astype(jnp.bfloat16), w2_ref[e],
                                            preferred_element_type=jnp.float32)

        with jax.named_scope("phase_scatter_start"):
            po[...] = partial.reshape(N_DEV, t_loc, d_model).astype(jnp.bfloat16)
            for d in range(1, N_DEV):
                dst = lax.rem(my + d, N_DEV)
                c = pltpu.make_async_remote_copy(
                    src_ref=po.at[d], dst_ref=og.at[N_DEV - d],
                    send_sem=o_send.at[d], recv_sem=o_recv.at[N_DEV - d],
                    device_id=(dst,), device_id_type=pl.DeviceIdType.MESH)
                c.start()
                sends.append(c)

        with jax.named_scope("phase_o_wait"):
            for d in range(1, N_DEV):
                wait_recv(og, o_send, o_recv, d)

        with jax.named_scope("phase_reduce_store"):
            acc = partial[0:t_loc, :]
            for d in range(1, N_DEV):
                acc = acc + og[d].astype(jnp.float32)
            out_ref[...] = acc

        with jax.named_scope("phase_send_drain"):
            for c in sends:
                c.wait_send()

    return pl.pallas_call(
        body,
        out_shape=jax.ShapeDtypeStruct((t_loc, d_model), jnp.float32),
        in_specs=[pl.BlockSpec(memory_space=pltpu.VMEM)] * 4,
        out_specs=pl.BlockSpec(memory_space=pltpu.VMEM),
        scratch_shapes=[
            pltpu.VMEM((N_DEV, t_loc, d_model), jnp.bfloat16),
            pltpu.VMEM((N_DEV, d_model, e_cols), jnp.float32),
            pltpu.VMEM((N_DEV, t_loc, n_exp), jnp.float32),
            pltpu.VMEM((N_DEV, t_loc, d_model), jnp.bfloat16),
            pltpu.VMEM((N_DEV, t_loc, d_model), jnp.bfloat16),
        ] + [pltpu.SemaphoreType.DMA((N_DEV,)) for _ in range(8)],
        compiler_params=pltpu.CompilerParams(collective_id=0),
    )(x, router, W1.astype(jnp.bfloat16), W2.astype(jnp.bfloat16))
